# baseline (speedup 1.0000x reference)
import jax
import jax.numpy as jnp
from jax.experimental import pallas as pl
from jax.experimental.pallas import tpu as pltpu

B, L, D = 2, 2048, 768
E = 8
TOPK = 2
H = 4 * D
L_HEAD = L // 3
L_WRIST = L // 3
L_PROPRIO = L - L_HEAD - L_WRIST
TL = 2048
NT = L // TL
HC = 2
Hc = H // HC


def _gate_kernel(x_ref, tc_ref, gw_ref, gb_ref, tw_ref, tb_ref,
                 eidx_ref, coef_ref):
    x = x_ref[...]
    sh = jnp.sum(x[:, :L_HEAD, :], axis=1)
    sw = jnp.sum(x[:, L_HEAD:L_HEAD + L_WRIST, :], axis=1)
    sp = jnp.sum(x[:, L_HEAD + L_WRIST:, :], axis=1)
    full = (sh + sw + sp) * (1.0 / L)
    hp = (sh + sp) * (1.0 / (L_HEAD + L_PROPRIO))
    wp = (sw + sp) * (1.0 / (L_WRIST + L_PROPRIO))
    gate_in = jnp.concatenate([full, hp, wp], axis=-1)
    logits = jnp.dot(gate_in, gw_ref[...],
                     preferred_element_type=jnp.float32) + gb_ref[...]
    tc = tc_ref[...]
    mod = jnp.dot(tc * jax.nn.sigmoid(tc), tw_ref[...],
                  preferred_element_type=jnp.float32) + tb_ref[...]
    scale = mod[:, :E]
    shift = mod[:, E:]
    logits = logits * (1.0 + scale) + shift
    m = jnp.max(logits, axis=-1, keepdims=True)
    ex = jnp.exp(logits - m)
    scores = ex / jnp.sum(ex, axis=-1, keepdims=True)
    iota = jax.lax.broadcasted_iota(jnp.int32, (B, E), 1)
    m1 = jnp.max(scores, axis=-1, keepdims=True)
    i1 = jnp.min(jnp.where(scores == m1, iota, E), axis=-1)
    masked = jnp.where(iota == i1[:, None], -jnp.inf, scores)
    m2 = jnp.max(masked, axis=-1, keepdims=True)
    i2 = jnp.min(jnp.where(masked == m2, iota, E), axis=-1)
    w1 = m1[:, 0]
    w2 = m2[:, 0]
    s = w1 + w2 + 1e-8
    w1 = w1 / s
    w2 = w2 / s
    eidx_ref[...] = jnp.stack([i1, i2, i2], axis=-1)
    coef_ref[...] = jnp.stack([w1, w2, jnp.ones_like(w1)], axis=-1)


def _tile_fully_in(t, lo, hi):
    return t * TL >= lo and (t + 1) * TL <= hi


def _moe_kernel(eidx_ref, coef_ref, x_ref, w1_ref, b1_ref, w2_ref, b2_ref,
                ws1_ref, bs1_ref, ws2_ref, bs2_ref, out_ref):
    b = pl.program_id(0)
    j = pl.program_id(1)
    hc = pl.program_id(2)
    coef = coef_ref[b, j]
    e = eidx_ref[b, j]
    pos = jax.lax.broadcasted_iota(jnp.int32, (TL, 1), 0)
    is_first = (j == 0) & (hc == 0)
    first_chunk = jnp.where(hc == 0, 1.0, 0.0)

    def run(w1, b1, w2, b2, is_expert):
        if is_expert:
            e1f = jnp.where(e == 1, 1.0, 0.0)
            e2f = jnp.where(e == 2, 1.0, 0.0)
        else:
            e1f = e2f = 0.0
        b2first = b2 * first_chunk
        for t in range(NT):
            xt = x_ref[0, pl.ds(t * TL, TL), :]
            h = jnp.dot(xt, w1, preferred_element_type=jnp.float32) + b1
            h = jax.nn.gelu(h, approximate=True).astype(jnp.bfloat16)
            part = jnp.dot(h, w2,
                           preferred_element_type=jnp.float32) + b2first
            if is_expert:
                p = pos + t * TL
                in_wrist = ((p >= L_HEAD) & (p < L_HEAD + L_WRIST)
                            ).astype(jnp.float32)
                in_head = (p < L_HEAD).astype(jnp.float32)
                keep = 1.0 - e1f * in_wrist - e2f * in_head
                val = (coef * keep) * part
            else:
                val = part

            @pl.when(is_first)
            def _():
                out_ref[0, pl.ds(t * TL, TL), :] = val

            @pl.when(jnp.logical_not(is_first))
            def _():
                out_ref[0, pl.ds(t * TL, TL), :] += val

    @pl.when(j < 2)
    def _():
        run(w1_ref[0], b1_ref[0, 0], w2_ref[0], b2_ref[0, 0], True)

    @pl.when(j == 2)
    def _():
        run(ws1_ref[...], bs1_ref[0], ws2_ref[...], bs2_ref[0], False)


def kernel(context_c, time_cond, gate_W, gate_b, time_W, time_b,
           W1, b1, W2, b2, Ws1, bs1, Ws2, bs2):
    eidx, coef = pl.pallas_call(
        _gate_kernel,
        out_shape=(
            jax.ShapeDtypeStruct((B, 3), jnp.int32),
            jax.ShapeDtypeStruct((B, 3), jnp.float32),
        ),
    )(context_c, time_cond, gate_W, gate_b, time_W, time_b)

    grid_spec = pltpu.PrefetchScalarGridSpec(
        num_scalar_prefetch=2,
        grid=(B, 3, HC),
        in_specs=[
            pl.BlockSpec((1, L, D), lambda b, j, hc, eidx, coef: (b, 0, 0),
                         pipeline_mode=pl.Buffered(buffer_count=1)),
            pl.BlockSpec((1, D, Hc),
                         lambda b, j, hc, eidx, coef: (eidx[b, j], 0, hc)),
            pl.BlockSpec((1, 1, Hc),
                         lambda b, j, hc, eidx, coef: (eidx[b, j], 0, hc)),
            pl.BlockSpec((1, Hc, D),
                         lambda b, j, hc, eidx, coef: (eidx[b, j], hc, 0)),
            pl.BlockSpec((1, 1, D),
                         lambda b, j, hc, eidx, coef: (eidx[b, j], 0, 0)),
            pl.BlockSpec((D, Hc), lambda b, j, hc, eidx, coef: (0, hc),
                         pipeline_mode=pl.Buffered(buffer_count=1)),
            pl.BlockSpec((1, Hc), lambda b, j, hc, eidx, coef: (0, hc)),
            pl.BlockSpec((Hc, D), lambda b, j, hc, eidx, coef: (hc, 0),
                         pipeline_mode=pl.Buffered(buffer_count=1)),
            pl.BlockSpec((1, D), lambda b, j, hc, eidx, coef: (0, 0)),
        ],
        out_specs=pl.BlockSpec((1, L, D),
                               lambda b, j, hc, eidx, coef: (b, 0, 0)),
    )
    out = pl.pallas_call(
        _moe_kernel,
        grid_spec=grid_spec,
        out_shape=jax.ShapeDtypeStruct((B, L, D), jnp.float32),
    )(eidx, coef, context_c, W1,
      b1.reshape(E, 1, H), W2, b2.reshape(E, 1, D),
      Ws1, bs1.reshape(1, H), Ws2, bs2.reshape(1, D))
    return out


# speedup vs baseline: 1.3420x; 1.3420x over previous
import jax
import jax.numpy as jnp
from jax.experimental import pallas as pl
from jax.experimental.pallas import tpu as pltpu

B, L, D = 2, 2048, 768
E = 8
TOPK = 2
H = 4 * D
L_HEAD = L // 3
L_WRIST = L // 3
L_PROPRIO = L - L_HEAD - L_WRIST
TL = 2048
NT = L // TL
HC = 3
Hc = H // HC


def _gate_kernel(x_ref, tc_ref, gw_ref, gb_ref, tw_ref, tb_ref,
                 eidx_ref, coef_ref):
    x = x_ref[...]
    sh = jnp.sum(x[:, :L_HEAD, :], axis=1)
    sw = jnp.sum(x[:, L_HEAD:L_HEAD + L_WRIST, :], axis=1)
    sp = jnp.sum(x[:, L_HEAD + L_WRIST:, :], axis=1)
    full = (sh + sw + sp) * (1.0 / L)
    hp = (sh + sp) * (1.0 / (L_HEAD + L_PROPRIO))
    wp = (sw + sp) * (1.0 / (L_WRIST + L_PROPRIO))
    gate_in = jnp.concatenate([full, hp, wp], axis=-1)
    logits = jnp.dot(gate_in, gw_ref[...],
                     preferred_element_type=jnp.float32) + gb_ref[...]
    tc = tc_ref[...]
    mod = jnp.dot(tc * jax.nn.sigmoid(tc), tw_ref[...],
                  preferred_element_type=jnp.float32) + tb_ref[...]
    scale = mod[:, :E]
    shift = mod[:, E:]
    logits = logits * (1.0 + scale) + shift
    m = jnp.max(logits, axis=-1, keepdims=True)
    ex = jnp.exp(logits - m)
    scores = ex / jnp.sum(ex, axis=-1, keepdims=True)
    iota = jax.lax.broadcasted_iota(jnp.int32, (B, E), 1)
    m1 = jnp.max(scores, axis=-1, keepdims=True)
    i1 = jnp.min(jnp.where(scores == m1, iota, E), axis=-1)
    masked = jnp.where(iota == i1[:, None], -jnp.inf, scores)
    m2 = jnp.max(masked, axis=-1, keepdims=True)
    i2 = jnp.min(jnp.where(masked == m2, iota, E), axis=-1)
    w1 = m1[:, 0]
    w2 = m2[:, 0]
    s = w1 + w2 + 1e-8
    w1 = w1 / s
    w2 = w2 / s
    eidx_ref[...] = jnp.stack([i1, i2, i2], axis=-1)
    coef_ref[...] = jnp.stack([w1, w2, jnp.ones_like(w1)], axis=-1)


def _tile_fully_in(t, lo, hi):
    return t * TL >= lo and (t + 1) * TL <= hi


def _moe_kernel(eidx_ref, coef_ref, x_ref, w1_ref, b1_ref, w2_ref, b2_ref,
                ws1_ref, bs1_ref, ws2_ref, bs2_ref, out_ref):
    b = pl.program_id(0)
    j = pl.program_id(1)
    hc = pl.program_id(2)
    coef = coef_ref[b, j]
    e = eidx_ref[b, j]
    pos = jax.lax.broadcasted_iota(jnp.int32, (TL, 1), 0)
    is_first = (j == 0) & (hc == 0)
    first_chunk = jnp.where(hc == 0, 1.0, 0.0)

    def run(w1, b1, w2, b2, is_expert):
        if is_expert:
            e1f = jnp.where(e == 1, 1.0, 0.0)
            e2f = jnp.where(e == 2, 1.0, 0.0)
        else:
            e1f = e2f = 0.0
        b2first = b2 * first_chunk
        for t in range(NT):
            xt = x_ref[0, pl.ds(t * TL, TL), :]
            h = jnp.dot(xt, w1, preferred_element_type=jnp.float32) + b1
            h = jax.nn.gelu(h, approximate=True).astype(jnp.bfloat16)
            part = jnp.dot(h, w2,
                           preferred_element_type=jnp.float32) + b2first
            if is_expert:
                p = pos + t * TL
                in_wrist = ((p >= L_HEAD) & (p < L_HEAD + L_WRIST)
                            ).astype(jnp.float32)
                in_head = (p < L_HEAD).astype(jnp.float32)
                keep = 1.0 - e1f * in_wrist - e2f * in_head
                val = (coef * keep) * part
            else:
                val = part

            @pl.when(is_first)
            def _():
                out_ref[0, pl.ds(t * TL, TL), :] = val

            @pl.when(jnp.logical_not(is_first))
            def _():
                out_ref[0, pl.ds(t * TL, TL), :] += val

    @pl.when(j < 2)
    def _():
        run(w1_ref[0], b1_ref[0, 0], w2_ref[0], b2_ref[0, 0], True)

    @pl.when(j == 2)
    def _():
        run(ws1_ref[...], bs1_ref[0], ws2_ref[...], bs2_ref[0], False)


def kernel(context_c, time_cond, gate_W, gate_b, time_W, time_b,
           W1, b1, W2, b2, Ws1, bs1, Ws2, bs2):
    eidx, coef = pl.pallas_call(
        _gate_kernel,
        out_shape=(
            jax.ShapeDtypeStruct((B, 3), jnp.int32),
            jax.ShapeDtypeStruct((B, 3), jnp.float32),
        ),
    )(context_c, time_cond, gate_W, gate_b, time_W, time_b)

    grid_spec = pltpu.PrefetchScalarGridSpec(
        num_scalar_prefetch=2,
        grid=(B, 3, HC),
        in_specs=[
            pl.BlockSpec((1, L, D), lambda b, j, hc, eidx, coef: (b, 0, 0)),
            pl.BlockSpec((1, D, Hc),
                         lambda b, j, hc, eidx, coef: (eidx[b, j], 0, hc)),
            pl.BlockSpec((1, 1, Hc),
                         lambda b, j, hc, eidx, coef: (eidx[b, j], 0, hc)),
            pl.BlockSpec((1, Hc, D),
                         lambda b, j, hc, eidx, coef: (eidx[b, j], hc, 0)),
            pl.BlockSpec((1, 1, D),
                         lambda b, j, hc, eidx, coef: (eidx[b, j], 0, 0)),
            pl.BlockSpec((D, Hc), lambda b, j, hc, eidx, coef: (0, hc)),
            pl.BlockSpec((1, Hc), lambda b, j, hc, eidx, coef: (0, hc)),
            pl.BlockSpec((Hc, D), lambda b, j, hc, eidx, coef: (hc, 0)),
            pl.BlockSpec((1, D), lambda b, j, hc, eidx, coef: (0, 0)),
        ],
        out_specs=pl.BlockSpec((1, L, D),
                               lambda b, j, hc, eidx, coef: (b, 0, 0)),
    )
    out = pl.pallas_call(
        _moe_kernel,
        grid_spec=grid_spec,
        out_shape=jax.ShapeDtypeStruct((B, L, D), jnp.float32),
    )(eidx, coef, context_c, W1,
      b1.reshape(E, 1, H), W2, b2.reshape(E, 1, D),
      Ws1, bs1.reshape(1, H), Ws2, bs2.reshape(1, D))
    return out
